# baseline (speedup 1.0000x reference)
import functools

import jax
import jax.numpy as jnp
from jax import lax
from jax.experimental import pallas as pl
from jax.experimental.pallas import tpu as pltpu
from jax.experimental.pallas import tpu_sc as plsc

NUM_EMBEDDINGS = 1000000
D = 64
BATCH = 16384
HIST = 20
B = BATCH * HIST

NC = 2
NS = 16
NW = NC * NS
BPW = B // NW

CW = 128
NCHUNK = BPW // CW
NBUF = 8

_mesh = plsc.VectorSubcoreMesh(
    core_axis_name="c", subcore_axis_name="s", num_cores=NC, num_subcores=NS
)


@functools.partial(
    pl.kernel,
    out_type=jax.ShapeDtypeStruct((B, D), jnp.float32),
    mesh=_mesh,
    compiler_params=pltpu.CompilerParams(use_tc_tiling_on_sc=False),
    scratch_types=[
        pltpu.VMEM((NCHUNK, CW), jnp.int32),
        pltpu.VMEM((NBUF, CW, D), jnp.float32),
        pltpu.SemaphoreType.DMA((NBUF,)),
        pltpu.SemaphoreType.DMA((NBUF,)),
    ],
)
def _gather_kernel(idx_hbm, table_hbm, out_hbm, idx_v, bufs, gsem, ssem):
    wid = lax.axis_index("s") * NC + lax.axis_index("c")
    row0 = wid * NCHUNK
    out0 = wid * BPW

    pltpu.sync_copy(idx_hbm.at[pl.ds(row0, NCHUNK)], idx_v)

    for b in range(NBUF):
        pltpu.async_copy(table_hbm.at[idx_v.at[b]], bufs.at[b], gsem.at[b])

    @pl.loop(0, NCHUNK, step=NBUF)
    def _round(i):
        for b in range(NBUF):
            c = i + b
            pltpu.make_async_copy(
                table_hbm.at[idx_v.at[c]], bufs.at[b], gsem.at[b]
            ).wait()
            dst = out_hbm.at[pl.ds(out0 + c * CW, CW)]
            pltpu.async_copy(bufs.at[b], dst, ssem.at[b])
            nc = c + NBUF

            @pl.when(nc < NCHUNK)
            def _refill():
                pltpu.make_async_copy(bufs.at[b], dst, ssem.at[b]).wait()
                pltpu.async_copy(
                    table_hbm.at[idx_v.at[nc]], bufs.at[b], gsem.at[b]
                )

    for b in range(NBUF):
        c = NCHUNK - NBUF + b
        pltpu.make_async_copy(
            bufs.at[b], out_hbm.at[pl.ds(out0 + c * CW, CW)], ssem.at[b]
        ).wait()


def kernel(token_ids, weight):
    idx = token_ids.reshape(B // CW, CW)
    out = _gather_kernel(idx, weight)
    return out.reshape(BATCH, HIST, D)
